# Initial kernel scaffold; baseline (speedup 1.0000x reference)
import jax
import jax.numpy as jnp
from jax.experimental import pallas as pl


def kernel(wi, cond, pdf, fac, W1, b1, W2, b2, W3, b3, W4, b4):
    raise NotImplementedError("write your pallas kernel here")


# speedup vs baseline: 7.3802x; 7.3802x over previous
import functools

import jax
import jax.numpy as jnp
import numpy as np
from jax import lax
from jax.experimental import pallas as pl
from jax.experimental.pallas import tpu as pltpu
from jax.experimental.pallas import tpu_sc as plsc

RES = 512
ANG = 16
D = 8
B = 16384
NC = 2
NS = 16
NW = NC * NS
ROWS_W = B // NW
PERW = D * ROWS_W
CHUNK = 128
LANES = 16


def _tc_body(cond_ref, wi_ref, a1_ref, c1_ref, a2_ref, c2_ref, a3_ref,
             c3_ref, a4_ref, c4_ref, fac_ref, idx_ref, w_ref):
    c = cond_ref[...]
    h = jnp.maximum(
        jnp.dot(a1_ref[...], c, preferred_element_type=jnp.float32)
        + c1_ref[...], 0.0)
    h = jnp.maximum(
        jnp.dot(a2_ref[...], h, preferred_element_type=jnp.float32)
        + c2_ref[...], 0.0)
    h = jnp.maximum(
        jnp.dot(a3_ref[...], h, preferred_element_type=jnp.float32)
        + c3_ref[...], 0.0)
    o = jnp.dot(a4_ref[...], h, preferred_element_type=jnp.float32) \
        + c4_ref[...]
    w = o[0:D]
    vx = o[D:2 * D]
    vy = o[2 * D:3 * D]
    zc = o[3 * D:4 * D]
    n = jnp.maximum(jnp.sqrt(vx * vx + vy * vy), 1e-12)
    cosv = vx / n
    sinv = vy / n
    z = 1.0 / (1.0 + jnp.exp(-zc))
    zi_f = jnp.clip(
        lax.round(z * ANG, lax.RoundingMethod.TO_NEAREST_EVEN),
        0.0, ANG - 1.0)
    z_idx = zi_f.astype(jnp.int32)
    facr = fac_ref[...]
    fg = jnp.zeros_like(w)
    for a in range(ANG):
        fg = jnp.where(z_idx == a, facr[:, a:a + 1], fg)
    wr = jnp.maximum(w, 0.0) * fg
    l1 = jnp.sum(jnp.abs(wr), axis=0, keepdims=True)
    wn = wr / jnp.maximum(l1, 1e-12)
    wix = wi_ref[0:1, :]
    wiy = wi_ref[1:2, :]
    wx = cosv * wix - sinv * wiy
    wy = sinv * wix + cosv * wiy
    fx = wx - jnp.floor(wx)
    fy = wy - jnp.floor(wy)
    ix = jnp.clip(jnp.floor(fx * RES), 0.0, RES - 1.0).astype(jnp.int32)
    iy = jnp.clip(jnp.floor(fy * RES), 0.0, RES - 1.0).astype(jnp.int32)
    dvec = lax.broadcasted_iota(jnp.int32, (D, ROWS_W), 0)
    flat = z_idx * (D * RES * RES) + dvec * (RES * RES) + iy * RES + ix
    idx_ref[0] = flat
    w_ref[0] = wn


def _sc_body(pdf_hbm, idx_hbm, w_hbm, out_hbm, idx_v, w_v, vals_v, out_v,
             sem):
    wid = lax.axis_index("s") * NC + lax.axis_index("c")
    pltpu.sync_copy(idx_hbm.at[wid], idx_v)
    pltpu.sync_copy(w_hbm.at[wid], w_v)

    def fire(j, carry):
        pltpu.async_copy(
            pdf_hbm.at[idx_v.at[pl.ds(j * CHUNK, CHUNK)]],
            vals_v.at[pl.ds(j * CHUNK, CHUNK)],
            sem,
        )
        return carry

    lax.fori_loop(0, PERW // CHUNK, fire, 0)
    pltpu.make_async_copy(pdf_hbm.at[idx_v], vals_v, sem).wait()

    def red(i, carry):
        base = i * LANES
        acc = w_v[pl.ds(base, LANES)] * vals_v[pl.ds(base, LANES)]
        for d in range(1, D):
            off = d * ROWS_W + base
            acc = acc + w_v[pl.ds(off, LANES)] * vals_v[pl.ds(off, LANES)]
        out_v[pl.ds(base, LANES)] = acc
        return carry

    lax.fori_loop(0, ROWS_W // LANES, red, 0)
    pltpu.sync_copy(out_v, out_hbm.at[pl.ds(wid * ROWS_W, ROWS_W)])


@functools.cache
def _sc_gather():
    mesh = plsc.VectorSubcoreMesh(core_axis_name="c", subcore_axis_name="s")
    return pl.kernel(
        _sc_body,
        mesh=mesh,
        out_type=jax.ShapeDtypeStruct((B,), jnp.float32),
        scratch_types=[
            pltpu.VMEM((PERW,), jnp.int32),
            pltpu.VMEM((PERW,), jnp.float32),
            pltpu.VMEM((PERW,), jnp.float32),
            pltpu.VMEM((ROWS_W,), jnp.float32),
            pltpu.SemaphoreType.DMA,
        ],
    )


_PERM = np.concatenate([
    np.arange(0, D),
    np.arange(D, 3 * D, 2),
    np.arange(D + 1, 3 * D, 2),
    np.arange(3 * D, 4 * D),
])


def _tc_call(cond_T, wi_T, a1, c1, a2, c2, a3, c3, a4, c4, fac_T):
    rep = lambda g: (0, 0)
    return pl.pallas_call(
        _tc_body,
        grid=(NW,),
        in_specs=[
            pl.BlockSpec((10, ROWS_W), lambda g: (0, g)),
            pl.BlockSpec((2, ROWS_W), lambda g: (0, g)),
            pl.BlockSpec((32, 10), rep),
            pl.BlockSpec((32, 1), rep),
            pl.BlockSpec((32, 32), rep),
            pl.BlockSpec((32, 1), rep),
            pl.BlockSpec((32, 32), rep),
            pl.BlockSpec((32, 1), rep),
            pl.BlockSpec((32, 32), rep),
            pl.BlockSpec((32, 1), rep),
            pl.BlockSpec((D, ANG), rep),
        ],
        out_specs=[
            pl.BlockSpec((1, D, ROWS_W), lambda g: (g, 0, 0)),
            pl.BlockSpec((1, D, ROWS_W), lambda g: (g, 0, 0)),
        ],
        out_shape=[
            jax.ShapeDtypeStruct((NW, D, ROWS_W), jnp.int32),
            jax.ShapeDtypeStruct((NW, D, ROWS_W), jnp.float32),
        ],
    )(cond_T, wi_T, a1, c1, a2, c2, a3, c3, a4, c4, fac_T)


def kernel(wi, cond, pdf, fac, W1, b1, W2, b2, W3, b3, W4, b4):
    cond_T = cond.T
    wi_T = wi.T
    fac_T = fac.T
    a1, c1 = W1.T, b1[:, None]
    a2, c2 = W2.T, b2[:, None]
    a3, c3 = W3.T, b3[:, None]
    a4, c4 = W4[:, _PERM].T, b4[_PERM][:, None]
    idx3, w3 = _tc_call(cond_T, wi_T, a1, c1, a2, c2, a3, c3, a4, c4, fac_T)
    idx_flat = idx3.reshape(NW, PERW)
    w_flat = w3.reshape(NW, PERW)
    pdf_flat = pdf.reshape(-1)
    return _sc_gather()(pdf_flat, idx_flat, w_flat)
